# baseline (speedup 1.0000x reference)
import functools

import jax
import jax.numpy as jnp
from jax import lax
from jax.experimental import pallas as pl
from jax.experimental.pallas import tpu as pltpu
from jax.experimental.pallas import tpu_sc as plsc


def _vtake(v, idx):
    dn = lax.GatherDimensionNumbers(
        offset_dims=(), collapsed_slice_dims=(0,), start_index_map=(0,))
    return lax.gather(v, idx[:, None], dn, slice_sizes=(1,),
                      mode=lax.GatherScatterMode.PROMISE_IN_BOUNDS)


C = 96
CP = 128
H = 224
W = 224
HW = H * W
K = 9
P = H * W
PK = P * K
NW = 32
CPW = PK // NW
NB = 112
NBLK = CPW // NB
LANES = 16
C_OUT = 96
KCP = K * CP


def _sc_bilinear_gather(table, cxy):
    mesh = plsc.VectorSubcoreMesh(core_axis_name="c", subcore_axis_name="s")

    buf_set = [
        pltpu.VMEM((2 * NB,), jnp.float32),
        pltpu.VMEM((4, NB), jnp.float32),
        pltpu.VMEM((NB,), jnp.int32),
        pltpu.VMEM((NB,), jnp.int32),
        pltpu.VMEM((NB,), jnp.int32),
        pltpu.VMEM((NB,), jnp.int32),
        pltpu.VMEM((NB, C), jnp.float32),
        pltpu.VMEM((NB, C), jnp.float32),
        pltpu.VMEM((NB, C), jnp.float32),
        pltpu.VMEM((NB, C), jnp.float32),
        pltpu.VMEM((NB, CP), jnp.float32),
        pltpu.SemaphoreType.DMA,
        pltpu.SemaphoreType.DMA,
        pltpu.SemaphoreType.DMA,
    ]
    NSET = len(buf_set)

    @functools.partial(
        pl.kernel,
        mesh=mesh,
        compiler_params=pltpu.CompilerParams(use_tc_tiling_on_sc=False,
                                             needs_layout_passes=False),
        out_type=jax.ShapeDtypeStruct((PK, CP), jnp.float32),
        scratch_types=buf_set + buf_set,
    )
    def sc_kernel(table_h, cxy_h, s_h, *bufs):
        cid = lax.axis_index("c")
        sid = lax.axis_index("s")
        wid = sid * 2 + cid
        base = wid * CPW
        sets = (bufs[:NSET], bufs[NSET:])
        lane = lax.iota(jnp.int32, LANES)
        ev_idx = (2 * lane) % LANES
        od_idx = (2 * lane + 1) % LANES
        lo_half = lane < (LANES // 2)

        def blk_off(b):
            return base + jnp.minimum(b, NBLK - 1) * NB

        def fire_cxy(b, st):
            cv = sets[st][0]
            sem_c = sets[st][13]
            pltpu.async_copy(cxy_h.at[pl.ds(2 * blk_off(b), 2 * NB)], cv,
                             sem_c)

        def wait_cxy(st):
            cv = sets[st][0]
            sem_c = sets[st][13]
            pltpu.make_async_copy(cxy_h.at[pl.ds(0, 2 * NB)], cv,
                                  sem_c).wait()

        def compute_idx(st):
            cv, wv, i0, i1, i2, i3 = sets[st][:6]
            for g in range(NB // LANES):
                va = cv[pl.ds(2 * LANES * g, LANES)]
                vb = cv[pl.ds(2 * LANES * g + LANES, LANES)]
                cxg = jnp.where(lo_half, _vtake(va, ev_idx),
                                _vtake(vb, ev_idx))
                cyg = jnp.where(lo_half, _vtake(va, od_idx),
                                _vtake(vb, od_idx))
                sl = pl.ds(g * LANES, LANES)
                x0 = cxg.astype(jnp.int32)
                y0 = cyg.astype(jnp.int32)
                fx = cxg - x0.astype(jnp.float32)
                fy = cyg - y0.astype(jnp.float32)
                gx = 1.0 - fx
                gy = 1.0 - fy
                x1 = x0 + 1
                y1 = y0 + 1
                vx1 = jnp.where(x1 < W, 1.0, 0.0)
                vy1 = jnp.where(y1 < H, 1.0, 0.0)
                x1c = jnp.minimum(x1, W - 1)
                y1c = jnp.minimum(y1, H - 1)
                wv[0, sl] = gx * gy
                wv[1, sl] = fx * gy * vx1
                wv[2, sl] = gx * fy * vy1
                wv[3, sl] = fx * fy * vx1 * vy1
                base00 = y0 * W
                base01 = y1c * W
                i0[sl] = base00 + x0
                i1[sl] = base00 + x1c
                i2[sl] = base01 + x0
                i3[sl] = base01 + x1c

        def fire_gathers(st):
            i0, i1, i2, i3, r0, r1, r2, r3 = sets[st][2:10]
            sem_g = sets[st][11]
            pltpu.async_copy(table_h.at[i0], r0, sem_g)
            pltpu.async_copy(table_h.at[i1], r1, sem_g)
            pltpu.async_copy(table_h.at[i2], r2, sem_g)
            pltpu.async_copy(table_h.at[i3], r3, sem_g)

        def wait_gathers(st):
            i0, i1, i2, i3, r0, r1, r2, r3 = sets[st][2:10]
            sem_g = sets[st][11]
            pltpu.make_async_copy(table_h.at[i0], r0, sem_g).wait()
            pltpu.make_async_copy(table_h.at[i1], r1, sem_g).wait()
            pltpu.make_async_copy(table_h.at[i2], r2, sem_g).wait()
            pltpu.make_async_copy(table_h.at[i3], r3, sem_g).wait()

        def weighted_sum(st):
            wv = sets[st][1]
            r0, r1, r2, r3, sv = sets[st][6:11]

            def grp(g, _):
                gsl = pl.ds(g * LANES, LANES)
                w0v = wv[0, gsl]
                w1v = wv[1, gsl]
                w2v = wv[2, gsl]
                w3v = wv[3, gsl]
                for j in range(LANES):
                    i = g * LANES + j
                    w0 = w0v[j]
                    w1 = w1v[j]
                    w2 = w2v[j]
                    w3 = w3v[j]
                    for c in range(C // LANES):
                        slc = pl.ds(c * LANES, LANES)
                        acc = (r0[i, slc] * w0 + r1[i, slc] * w1
                               + r2[i, slc] * w2 + r3[i, slc] * w3)
                        sv[i, slc] = acc
                return 0

            lax.fori_loop(0, NB // LANES, grp, 0)

        def fire_store(b, st):
            sv = sets[st][10]
            sem_s = sets[st][12]
            pltpu.async_copy(sv, s_h.at[pl.ds(blk_off(b), NB)], sem_s)

        def wait_store(st):
            sv = sets[st][10]
            sem_s = sets[st][12]
            pltpu.make_async_copy(sv, s_h.at[pl.ds(base, NB)], sem_s).wait()

        zeros = jnp.zeros((LANES,), jnp.float32)
        for st in range(2):
            sv = sets[st][10]

            def zpad(i, _, sv=sv):
                sv[i, pl.ds(C, LANES)] = zeros
                sv[i, pl.ds(C + LANES, LANES)] = zeros
                return 0

            lax.fori_loop(0, NB, zpad, 0)

        pltpu.sync_copy(cxy_h.at[pl.ds(2 * base, 2 * NB)], sets[0][0])
        compute_idx(0)
        fire_gathers(0)
        fire_cxy(1, 1)
        fire_cxy(2, 0)

        def pair_body(t, _):
            b0 = 2 * t
            wait_cxy(1)
            compute_idx(1)
            fire_gathers(1)
            fire_cxy(b0 + 3, 1)
            wait_gathers(0)

            @pl.when(t > 0)
            def _():
                wait_store(0)

            weighted_sum(0)
            fire_store(b0, 0)
            wait_cxy(0)
            compute_idx(0)
            fire_gathers(0)
            fire_cxy(b0 + 4, 0)
            wait_gathers(1)

            @pl.when(t > 0)
            def _():
                wait_store(1)

            weighted_sum(1)
            fire_store(b0 + 1, 1)
            return 0

        lax.fori_loop(0, NBLK // 2, pair_body, 0)
        wait_gathers(0)
        wait_store(0)
        wait_store(1)
        wait_cxy(0)
        wait_cxy(1)

    return sc_kernel(table, cxy)


def _tc_transpose(x2):
    BLK = 1024

    def body(x_ref, o_ref):
        o_ref[...] = x_ref[...].T

    return pl.pallas_call(
        body,
        grid=(HW // BLK,),
        in_specs=[pl.BlockSpec((C, BLK), lambda i: (0, i))],
        out_specs=pl.BlockSpec((BLK, C), lambda i: (i, 0)),
        out_shape=jax.ShapeDtypeStruct((HW, C), jnp.float32),
    )(x2)


def _tc_contract(s2, w2, bias2):
    PB = 1024

    def body(s_ref, w_ref, b_ref, o_ref):
        o = lax.dot_general(w_ref[...], s_ref[...],
                            (((1,), (1,)), ((), ())),
                            preferred_element_type=jnp.float32)
        o_ref[...] = o + b_ref[...]

    return pl.pallas_call(
        body,
        grid=(P // PB,),
        in_specs=[
            pl.BlockSpec((PB, KCP), lambda i: (i, 0)),
            pl.BlockSpec((C_OUT, KCP), lambda i: (0, 0)),
            pl.BlockSpec((C_OUT, 1), lambda i: (0, 0)),
        ],
        out_specs=pl.BlockSpec((C_OUT, PB), lambda i: (0, i)),
        out_shape=jax.ShapeDtypeStruct((C_OUT, P), jnp.float32),
    )(s2, w2, bias2)


def kernel(x, sample_map, weight, bias):
    table = _tc_transpose(x.reshape(C, HW))
    cxy = sample_map.reshape(2 * PK)
    s = _sc_bilinear_gather(table, cxy)
    s2 = s.reshape(P, KCP)
    w3 = jnp.transpose(weight, (0, 2, 1))
    w2 = jnp.pad(w3, ((0, 0), (0, 0), (0, CP - C))).reshape(C_OUT, KCP)
    out = _tc_contract(s2, w2, bias.reshape(C_OUT, 1))
    return out.reshape(1, C_OUT, H, W)


# speedup vs baseline: 1.6079x; 1.6079x over previous
import functools

import jax
import jax.numpy as jnp
from jax import lax
from jax.experimental import pallas as pl
from jax.experimental.pallas import tpu as pltpu
from jax.experimental.pallas import tpu_sc as plsc

C = 96
CP = 128
H = 224
W = 224
HW = H * W
K = 9
P = H * W
PK = P * K
NW = 32
CPW = PK // NW
NB = 48
NBLK = CPW // NB
LANES = 16
C_OUT = 96


def _sc_bilinear_gather(table, cx, cy):
    mesh = plsc.VectorSubcoreMesh(core_axis_name="c", subcore_axis_name="s")

    buf_set = [
        pltpu.VMEM((NB,), jnp.float32),
        pltpu.VMEM((NB,), jnp.float32),
        pltpu.VMEM((4, NB), jnp.float32),
        pltpu.VMEM((NB,), jnp.int32),
        pltpu.VMEM((NB,), jnp.int32),
        pltpu.VMEM((NB,), jnp.int32),
        pltpu.VMEM((NB,), jnp.int32),
        pltpu.VMEM((NB, CP), jnp.float32),
        pltpu.VMEM((NB, CP), jnp.float32),
        pltpu.VMEM((NB, CP), jnp.float32),
        pltpu.VMEM((NB, CP), jnp.float32),
        pltpu.VMEM((NB, CP), jnp.float32),
        pltpu.SemaphoreType.DMA,
        pltpu.SemaphoreType.DMA,
        pltpu.SemaphoreType.DMA,
    ]
    NSET = len(buf_set)

    @functools.partial(
        pl.kernel,
        mesh=mesh,
        compiler_params=pltpu.CompilerParams(use_tc_tiling_on_sc=True),
        out_type=jax.ShapeDtypeStruct((PK, CP), jnp.float32),
        scratch_types=buf_set + buf_set,
    )
    def sc_kernel(table_h, cx_h, cy_h, s_h, *bufs):
        cid = lax.axis_index("c")
        sid = lax.axis_index("s")
        wid = sid * 2 + cid
        base = wid * CPW
        sets = (bufs[:NSET], bufs[NSET:])

        def blk_off(b):
            return base + jnp.minimum(b, NBLK - 1) * NB

        def fire_cxy(b, st):
            cxv, cyv = sets[st][0], sets[st][1]
            sem_c = sets[st][14]
            off = blk_off(b)
            pltpu.async_copy(cx_h.at[pl.ds(off, NB)], cxv, sem_c)
            pltpu.async_copy(cy_h.at[pl.ds(off, NB)], cyv, sem_c)

        def wait_cxy(st):
            cxv, cyv = sets[st][0], sets[st][1]
            sem_c = sets[st][14]
            pltpu.make_async_copy(cx_h.at[pl.ds(0, NB)], cxv, sem_c).wait()
            pltpu.make_async_copy(cy_h.at[pl.ds(0, NB)], cyv, sem_c).wait()

        def compute_idx(st):
            cxv, cyv, wv, i0, i1, i2, i3 = sets[st][:7]
            for g in range(NB // LANES):
                sl = pl.ds(g * LANES, LANES)
                cxg = cxv[sl]
                cyg = cyv[sl]
                x0 = cxg.astype(jnp.int32)
                y0 = cyg.astype(jnp.int32)
                fx = cxg - x0.astype(jnp.float32)
                fy = cyg - y0.astype(jnp.float32)
                gx = 1.0 - fx
                gy = 1.0 - fy
                x1 = x0 + 1
                y1 = y0 + 1
                vx1 = jnp.where(x1 < W, 1.0, 0.0)
                vy1 = jnp.where(y1 < H, 1.0, 0.0)
                x1c = jnp.minimum(x1, W - 1)
                y1c = jnp.minimum(y1, H - 1)
                wv[0, sl] = gx * gy
                wv[1, sl] = fx * gy * vx1
                wv[2, sl] = gx * fy * vy1
                wv[3, sl] = fx * fy * vx1 * vy1
                base00 = y0 * W
                base01 = y1c * W
                i0[sl] = base00 + x0
                i1[sl] = base00 + x1c
                i2[sl] = base01 + x0
                i3[sl] = base01 + x1c

        def fire_gathers(st):
            i0, i1, i2, i3, r0, r1, r2, r3 = sets[st][3:11]
            sem_g = sets[st][12]
            pltpu.async_copy(table_h.at[i0], r0, sem_g)
            pltpu.async_copy(table_h.at[i1], r1, sem_g)
            pltpu.async_copy(table_h.at[i2], r2, sem_g)
            pltpu.async_copy(table_h.at[i3], r3, sem_g)

        def wait_gathers(st):
            i0, i1, i2, i3, r0, r1, r2, r3 = sets[st][3:11]
            sem_g = sets[st][12]
            pltpu.make_async_copy(table_h.at[i0], r0, sem_g).wait()
            pltpu.make_async_copy(table_h.at[i1], r1, sem_g).wait()
            pltpu.make_async_copy(table_h.at[i2], r2, sem_g).wait()
            pltpu.make_async_copy(table_h.at[i3], r3, sem_g).wait()

        def weighted_sum(st):
            wv = sets[st][2]
            r0, r1, r2, r3, sv = sets[st][7:12]

            def grp(g, _):
                gsl = pl.ds(g * LANES, LANES)
                w0v = wv[0, gsl]
                w1v = wv[1, gsl]
                w2v = wv[2, gsl]
                w3v = wv[3, gsl]
                for j in range(LANES):
                    i = g * LANES + j
                    w0 = w0v[j]
                    w1 = w1v[j]
                    w2 = w2v[j]
                    w3 = w3v[j]
                    for c in range(C // LANES):
                        slc = pl.ds(c * LANES, LANES)
                        acc = (r0[i, slc] * w0 + r1[i, slc] * w1
                               + r2[i, slc] * w2 + r3[i, slc] * w3)
                        sv[i, slc] = acc
                return 0

            lax.fori_loop(0, NB // LANES, grp, 0)

        def fire_store(b, st):
            sv = sets[st][11]
            sem_s = sets[st][13]
            pltpu.async_copy(sv, s_h.at[pl.ds(blk_off(b), NB)], sem_s)

        def wait_store(st):
            sv = sets[st][11]
            sem_s = sets[st][13]
            pltpu.make_async_copy(sv, s_h.at[pl.ds(base, NB)], sem_s).wait()

        zeros = jnp.zeros((LANES,), jnp.float32)
        for st in range(2):
            sv = sets[st][11]

            def zpad(i, _, sv=sv):
                sv[i, pl.ds(C, LANES)] = zeros
                sv[i, pl.ds(C + LANES, LANES)] = zeros
                return 0

            lax.fori_loop(0, NB, zpad, 0)

        pltpu.sync_copy(cx_h.at[pl.ds(base, NB)], sets[0][0])
        pltpu.sync_copy(cy_h.at[pl.ds(base, NB)], sets[0][1])
        compute_idx(0)
        fire_gathers(0)
        fire_cxy(1, 1)
        fire_cxy(2, 0)

        def pair_body(t, _):
            b0 = 2 * t
            wait_cxy(1)
            compute_idx(1)
            fire_gathers(1)
            fire_cxy(b0 + 3, 1)
            wait_gathers(0)

            @pl.when(t > 0)
            def _():
                wait_store(0)

            weighted_sum(0)
            fire_store(b0, 0)
            wait_cxy(0)
            compute_idx(0)
            fire_gathers(0)
            fire_cxy(b0 + 4, 0)
            wait_gathers(1)

            @pl.when(t > 0)
            def _():
                wait_store(1)

            weighted_sum(1)
            fire_store(b0 + 1, 1)
            return 0

        lax.fori_loop(0, NBLK // 2, pair_body, 0)
        wait_gathers(0)
        wait_store(0)
        wait_store(1)
        wait_cxy(0)
        wait_cxy(1)

    return sc_kernel(table, cx, cy).reshape(K, P, CP)


def _tc_transpose_pad(x2):
    BLK = 1024

    def body(x_ref, o_ref):
        o_ref[...] = jnp.concatenate(
            [x_ref[...].T, jnp.zeros((BLK, CP - C), jnp.float32)], axis=1)

    return pl.pallas_call(
        body,
        grid=(HW // BLK,),
        in_specs=[pl.BlockSpec((C, BLK), lambda i: (0, i))],
        out_specs=pl.BlockSpec((BLK, CP), lambda i: (i, 0)),
        out_shape=jax.ShapeDtypeStruct((HW, CP), jnp.float32),
    )(x2)


def _tc_contract(s3, w3, bias2):
    PB = 1024

    def body(s_ref, w_ref, b_ref, o_ref):
        k = pl.program_id(1)
        o = lax.dot_general(w_ref[0], s_ref[0],
                            (((1,), (1,)), ((), ())),
                            preferred_element_type=jnp.float32)

        @pl.when(k == 0)
        def _():
            o_ref[...] = o + b_ref[...]

        @pl.when(k > 0)
        def _():
            o_ref[...] = o_ref[...] + o

    return pl.pallas_call(
        body,
        grid=(P // PB, K),
        in_specs=[
            pl.BlockSpec((1, PB, CP), lambda i, k: (k, i, 0)),
            pl.BlockSpec((1, C_OUT, CP), lambda i, k: (k, 0, 0)),
            pl.BlockSpec((C_OUT, 1), lambda i, k: (0, 0)),
        ],
        out_specs=pl.BlockSpec((C_OUT, PB), lambda i, k: (0, i)),
        out_shape=jax.ShapeDtypeStruct((C_OUT, P), jnp.float32),
    )(s3, w3, bias2)


def kernel(x, sample_map, weight, bias):
    table = _tc_transpose_pad(x.reshape(C, HW))
    sm = sample_map.reshape(P, K, 2)
    cx = jnp.transpose(sm[:, :, 0], (1, 0)).reshape(PK)
    cy = jnp.transpose(sm[:, :, 1], (1, 0)).reshape(PK)
    s3 = _sc_bilinear_gather(table, cx, cy)
    w3 = jnp.pad(jnp.transpose(weight, (2, 0, 1)),
                 ((0, 0), (0, 0), (0, CP - C)))
    out = _tc_contract(s3, w3, bias.reshape(C_OUT, 1))
    return out.reshape(1, C_OUT, H, W)


# speedup vs baseline: 2.1105x; 1.3126x over previous
import functools

import jax
import jax.numpy as jnp
from jax import lax
from jax.experimental import pallas as pl
from jax.experimental.pallas import tpu as pltpu
from jax.experimental.pallas import tpu_sc as plsc

C = 96
CP = 128
H = 224
W = 224
HW = H * W
K = 9
P = H * W
PK = P * K
NW = 32
CPW = PK // NW
NB = 48
NBLK = CPW // NB
LANES = 16
C_OUT = 96


def _sc_bilinear_gather(table, cx, cy):
    mesh = plsc.VectorSubcoreMesh(core_axis_name="c", subcore_axis_name="s")

    buf_set = [
        pltpu.VMEM((NB,), jnp.float32),
        pltpu.VMEM((NB,), jnp.float32),
        pltpu.VMEM((4, NB), jnp.float32),
        pltpu.VMEM((NB,), jnp.int32),
        pltpu.VMEM((NB,), jnp.int32),
        pltpu.VMEM((NB,), jnp.int32),
        pltpu.VMEM((NB,), jnp.int32),
        pltpu.VMEM((NB, CP), jnp.float32),
        pltpu.VMEM((NB, CP), jnp.float32),
        pltpu.VMEM((NB, CP), jnp.float32),
        pltpu.VMEM((NB, CP), jnp.float32),
        pltpu.VMEM((NB, CP), jnp.float32),
        pltpu.SemaphoreType.DMA,
        pltpu.SemaphoreType.DMA,
        pltpu.SemaphoreType.DMA,
    ]
    NSET = len(buf_set)

    @functools.partial(
        pl.kernel,
        mesh=mesh,
        compiler_params=pltpu.CompilerParams(use_tc_tiling_on_sc=True),
        out_type=jax.ShapeDtypeStruct((PK, CP), jnp.float32),
        scratch_types=buf_set + buf_set,
    )
    def sc_kernel(table_h, cx_h, cy_h, s_h, *bufs):
        cid = lax.axis_index("c")
        sid = lax.axis_index("s")
        wid = sid * 2 + cid
        base = wid * CPW
        sets = (bufs[:NSET], bufs[NSET:])

        def blk_off(b):
            return base + jnp.minimum(b, NBLK - 1) * NB

        def fire_cxy(b, st):
            cxv, cyv = sets[st][0], sets[st][1]
            sem_c = sets[st][14]
            off = blk_off(b)
            pltpu.async_copy(cx_h.at[pl.ds(off, NB)], cxv, sem_c)
            pltpu.async_copy(cy_h.at[pl.ds(off, NB)], cyv, sem_c)

        def wait_cxy(st):
            cxv, cyv = sets[st][0], sets[st][1]
            sem_c = sets[st][14]
            pltpu.make_async_copy(cx_h.at[pl.ds(0, NB)], cxv, sem_c).wait()
            pltpu.make_async_copy(cy_h.at[pl.ds(0, NB)], cyv, sem_c).wait()

        def compute_idx(st):
            cxv, cyv, wv, i0, i1, i2, i3 = sets[st][:7]
            for g in range(NB // LANES):
                sl = pl.ds(g * LANES, LANES)
                cxg = cxv[sl]
                cyg = cyv[sl]
                x0 = cxg.astype(jnp.int32)
                y0 = cyg.astype(jnp.int32)
                fx = cxg - x0.astype(jnp.float32)
                fy = cyg - y0.astype(jnp.float32)
                gx = 1.0 - fx
                gy = 1.0 - fy
                x1 = x0 + 1
                y1 = y0 + 1
                vx1 = jnp.where(x1 < W, 1.0, 0.0)
                vy1 = jnp.where(y1 < H, 1.0, 0.0)
                x1c = jnp.minimum(x1, W - 1)
                y1c = jnp.minimum(y1, H - 1)
                wv[0, sl] = gx * gy
                wv[1, sl] = fx * gy * vx1
                wv[2, sl] = gx * fy * vy1
                wv[3, sl] = fx * fy * vx1 * vy1
                base00 = y0 * W
                base01 = y1c * W
                i0[sl] = base00 + x0
                i1[sl] = base00 + x1c
                i2[sl] = base01 + x0
                i3[sl] = base01 + x1c

        def fire_gathers(st):
            i0, i1, i2, i3, r0, r1, r2, r3 = sets[st][3:11]
            sem_g = sets[st][12]
            pltpu.async_copy(table_h.at[i0], r0, sem_g)
            pltpu.async_copy(table_h.at[i1], r1, sem_g)
            pltpu.async_copy(table_h.at[i2], r2, sem_g)
            pltpu.async_copy(table_h.at[i3], r3, sem_g)

        def wait_gathers(st):
            i0, i1, i2, i3, r0, r1, r2, r3 = sets[st][3:11]
            sem_g = sets[st][12]
            pltpu.make_async_copy(table_h.at[i0], r0, sem_g).wait()
            pltpu.make_async_copy(table_h.at[i1], r1, sem_g).wait()
            pltpu.make_async_copy(table_h.at[i2], r2, sem_g).wait()
            pltpu.make_async_copy(table_h.at[i3], r3, sem_g).wait()

        def weighted_sum(st):
            wv = sets[st][2]
            r0, r1, r2, r3, sv = sets[st][7:12]

            def grp(g, _):
                gsl = pl.ds(g * LANES, LANES)
                w0v = wv[0, gsl]
                w1v = wv[1, gsl]
                w2v = wv[2, gsl]
                w3v = wv[3, gsl]
                for j in range(LANES):
                    i = g * LANES + j
                    w0 = w0v[j]
                    w1 = w1v[j]
                    w2 = w2v[j]
                    w3 = w3v[j]
                    for c in range(C // LANES):
                        slc = pl.ds(c * LANES, LANES)
                        acc = (r0[i, slc] * w0 + r1[i, slc] * w1
                               + r2[i, slc] * w2 + r3[i, slc] * w3)
                        sv[i, slc] = acc
                return 0

            lax.fori_loop(0, NB // LANES, grp, 0)

        def fire_store(b, st):
            sv = sets[st][11]
            sem_s = sets[st][13]
            pltpu.async_copy(sv, s_h.at[pl.ds(blk_off(b), NB)], sem_s)

        def wait_store(st):
            sv = sets[st][11]
            sem_s = sets[st][13]
            pltpu.make_async_copy(sv, s_h.at[pl.ds(base, NB)], sem_s).wait()

        zeros = jnp.zeros((LANES,), jnp.float32)
        for st in range(2):
            sv = sets[st][11]

            def zpad(i, _, sv=sv):
                sv[i, pl.ds(C, LANES)] = zeros
                sv[i, pl.ds(C + LANES, LANES)] = zeros
                return 0

            lax.fori_loop(0, NB, zpad, 0)

        pltpu.sync_copy(cx_h.at[pl.ds(base, NB)], sets[0][0])
        pltpu.sync_copy(cy_h.at[pl.ds(base, NB)], sets[0][1])
        compute_idx(0)
        fire_gathers(0)
        fire_cxy(1, 1)
        fire_cxy(2, 0)

        def pair_body(t, _):
            b0 = 2 * t
            wait_cxy(1)
            compute_idx(1)
            fire_gathers(1)
            fire_cxy(b0 + 3, 1)
            wait_gathers(0)

            @pl.when(t > 0)
            def _():
                wait_store(0)

            weighted_sum(0)
            fire_store(b0, 0)
            wait_cxy(0)
            compute_idx(0)
            fire_gathers(0)
            fire_cxy(b0 + 4, 0)
            wait_gathers(1)

            @pl.when(t > 0)
            def _():
                wait_store(1)

            weighted_sum(1)
            fire_store(b0 + 1, 1)
            return 0

        lax.fori_loop(0, NBLK // 2, pair_body, 0)
        wait_gathers(0)
        wait_store(0)
        wait_store(1)
        wait_cxy(0)
        wait_cxy(1)

    return sc_kernel(table, cx, cy).reshape(K, P, CP)


def _tc_transpose_pad(x2):
    BLK = 1024

    def body(x_ref, o_ref):
        o_ref[...] = jnp.concatenate(
            [x_ref[...].T, jnp.zeros((BLK, CP - C), jnp.float32)], axis=1)

    return pl.pallas_call(
        body,
        grid=(HW // BLK,),
        in_specs=[pl.BlockSpec((C, BLK), lambda i: (0, i))],
        out_specs=pl.BlockSpec((BLK, CP), lambda i: (i, 0)),
        out_shape=jax.ShapeDtypeStruct((HW, CP), jnp.float32),
    )(x2)


def _tc_contract(s3, w3, bias2):
    PB = 1024

    def body(s_ref, w_ref, b_ref, o_ref):
        o = b_ref[...]
        for k in range(K):
            o = o + lax.dot_general(w_ref[k], s_ref[k],
                                    (((1,), (1,)), ((), ())),
                                    preferred_element_type=jnp.float32)
        o_ref[...] = o

    return pl.pallas_call(
        body,
        grid=(P // PB,),
        in_specs=[
            pl.BlockSpec((K, PB, CP), lambda i: (0, i, 0)),
            pl.BlockSpec((K, C_OUT, CP), lambda i: (0, 0, 0)),
            pl.BlockSpec((C_OUT, 1), lambda i: (0, 0)),
        ],
        out_specs=pl.BlockSpec((C_OUT, PB), lambda i: (0, i)),
        out_shape=jax.ShapeDtypeStruct((C_OUT, P), jnp.float32),
    )(s3, w3, bias2)


def kernel(x, sample_map, weight, bias):
    table = _tc_transpose_pad(x.reshape(C, HW))
    sm = sample_map.reshape(P, K, 2)
    cx = jnp.transpose(sm[:, :, 0], (1, 0)).reshape(PK)
    cy = jnp.transpose(sm[:, :, 1], (1, 0)).reshape(PK)
    s3 = _sc_bilinear_gather(table, cx, cy)
    w3 = jnp.pad(jnp.transpose(weight, (2, 0, 1)),
                 ((0, 0), (0, 0), (0, CP - C)))
    out = _tc_contract(s3, w3, bias.reshape(C_OUT, 1))
    return out.reshape(1, C_OUT, H, W)
